# baseline (speedup 1.0000x reference)
import functools

import jax
import jax.numpy as jnp
from jax import lax
from jax.experimental import pallas as pl
from jax.experimental.pallas import tpu as pltpu
from jax.experimental.pallas import tpu_sc as plsc


_NC = 2
_NS = 16
_NW = _NC * _NS
_CH = 64
_LANES = 16


def _make_sc_add(b, l, d):
    lpw = l // _NW
    nch = lpw // _CH
    cw = _CH * d
    mesh = plsc.VectorSubcoreMesh(core_axis_name="c", subcore_axis_name="s")

    @functools.partial(
        pl.kernel,
        out_type=jax.ShapeDtypeStruct((b, l * d), jnp.float32),
        mesh=mesh,
        scratch_types=[
            pltpu.VMEM((cw,), jnp.float32),
            pltpu.VMEM((cw,), jnp.float32),
        ],
    )
    def sc_add(x_hbm, table_hbm, out_hbm, buf, tbuf):
        cid = lax.axis_index("c")
        sid = lax.axis_index("s")
        wid = cid * _NS + sid
        tbase = wid * lpw * d

        def chunk_body(i, _):
            e0 = tbase + i * cw
            pltpu.sync_copy(table_hbm.at[pl.ds(e0, cw)], tbuf)

            def batch_body(bi, _):
                pltpu.sync_copy(x_hbm.at[bi].at[pl.ds(e0, cw)], buf)

                @plsc.parallel_loop(0, cw, step=_LANES, unroll=8)
                def add_body(j):
                    plsc.addupdate(buf.at[pl.ds(j, _LANES)],
                                   tbuf[pl.ds(j, _LANES)])

                pltpu.sync_copy(buf, out_hbm.at[bi].at[pl.ds(e0, cw)])
                return 0

            lax.fori_loop(0, b, batch_body, 0)
            return 0

        lax.fori_loop(0, nch, chunk_body, 0)

    return sc_add


def kernel(x, table):
    B, L, D = x.shape
    out = _make_sc_add(B, L, D)(x.reshape(B, L * D), table.reshape(L * D))
    return out.reshape(B, L, D)


# speedup vs baseline: 1.1756x; 1.1756x over previous
import functools

import jax
import jax.numpy as jnp
from jax import lax
from jax.experimental import pallas as pl
from jax.experimental.pallas import tpu as pltpu
from jax.experimental.pallas import tpu_sc as plsc


_NC = 2
_NS = 16
_NW = _NC * _NS
_CH = 32
_LANES = 16
_UNROLL = 16


def _make_sc_add(b, l, d):
    lpw = l // _NW
    nch = lpw // _CH
    cw = _CH * d
    nsteps = nch * b
    steps = [(i, bi) for i in range(nch) for bi in range(b)]
    mesh = plsc.VectorSubcoreMesh(core_axis_name="c", subcore_axis_name="s")

    @functools.partial(
        pl.kernel,
        out_type=jax.ShapeDtypeStruct((b, l * d), jnp.float32),
        mesh=mesh,
        scratch_types=[
            pltpu.VMEM((cw,), jnp.float32),
            pltpu.VMEM((cw,), jnp.float32),
            pltpu.VMEM((cw,), jnp.float32),
            pltpu.VMEM((cw,), jnp.float32),
            pltpu.SemaphoreType.DMA,
            pltpu.SemaphoreType.DMA,
            pltpu.SemaphoreType.DMA,
            pltpu.SemaphoreType.DMA,
            pltpu.SemaphoreType.DMA,
            pltpu.SemaphoreType.DMA,
        ],
    )
    def sc_add(x_hbm, table_hbm, out_hbm,
               xb0, xb1, tb0, tb1, sx0, sx1, st0, st1, so0, so1):
        bufs = (xb0, xb1)
        tbufs = (tb0, tb1)
        sxs = (sx0, sx1)
        sts = (st0, st1)
        sos = (so0, so1)
        cid = lax.axis_index("c")
        sid = lax.axis_index("s")
        wid = cid * _NS + sid
        tbase = wid * lpw * d

        def t_off(i):
            return tbase + i * cw

        tdesc = [None] * (nch + 1)
        xdesc = [None] * nsteps
        odesc = [None] * nsteps

        tdesc[0] = pltpu.async_copy(
            table_hbm.at[pl.ds(t_off(0), cw)], tbufs[0], sts[0])
        xdesc[0] = pltpu.async_copy(
            x_hbm.at[0].at[pl.ds(t_off(0), cw)], bufs[0], sxs[0])

        for s, (i, bi) in enumerate(steps):
            xb = bufs[s % 2]
            if bi == 0:
                tdesc[i].wait()
                if i + 1 < nch:
                    tdesc[i + 1] = pltpu.async_copy(
                        table_hbm.at[pl.ds(t_off(i + 1), cw)],
                        tbufs[(i + 1) % 2], sts[(i + 1) % 2])
            xdesc[s].wait()
            if s + 1 < nsteps:
                ni, nbi = steps[s + 1]
                if s >= 1:
                    odesc[s - 1].wait()
                xdesc[s + 1] = pltpu.async_copy(
                    x_hbm.at[nbi].at[pl.ds(t_off(ni), cw)],
                    bufs[(s + 1) % 2], sxs[(s + 1) % 2])
            tb = tbufs[i % 2]

            @plsc.parallel_loop(0, cw, step=_LANES, unroll=_UNROLL)
            def add_body(j, xb=xb, tb=tb):
                plsc.addupdate(xb.at[pl.ds(j, _LANES)], tb[pl.ds(j, _LANES)])

            odesc[s] = pltpu.async_copy(
                xb, out_hbm.at[bi].at[pl.ds(t_off(i), cw)], sos[s % 2])

        odesc[nsteps - 2].wait()
        odesc[nsteps - 1].wait()

    return sc_add


def kernel(x, table):
    B, L, D = x.shape
    out = _make_sc_add(B, L, D)(x.reshape(B, L * D), table.reshape(L * D))
    return out.reshape(B, L, D)


# speedup vs baseline: 2.7495x; 2.3388x over previous
import functools

import jax
import jax.numpy as jnp
from jax import lax
from jax.experimental import pallas as pl
from jax.experimental.pallas import tpu as pltpu
from jax.experimental.pallas import tpu_sc as plsc


_NC = 2
_NS = 16
_NW = _NC * _NS
_CH = 32
_LANES = 16


def _make_sc_add(b, l, d):
    lpw = l // _NW
    nch = lpw // _CH
    nsteps = nch * b
    nvec = d // _LANES
    mesh = plsc.VectorSubcoreMesh(core_axis_name="c", subcore_axis_name="s")

    @functools.partial(
        pl.kernel,
        out_type=jax.ShapeDtypeStruct((b * l, d), jnp.float32),
        mesh=mesh,
        scratch_types=[
            pltpu.VMEM((_CH, d), jnp.float32),
            pltpu.VMEM((_CH, d), jnp.float32),
            pltpu.VMEM((_CH, d), jnp.float32),
            pltpu.VMEM((_CH, d), jnp.float32),
            pltpu.SemaphoreType.DMA,
            pltpu.SemaphoreType.DMA,
            pltpu.SemaphoreType.DMA,
            pltpu.SemaphoreType.DMA,
            pltpu.SemaphoreType.DMA,
            pltpu.SemaphoreType.DMA,
        ],
    )
    def sc_add(x_hbm, table_hbm, out_hbm,
               xb0, xb1, tb0, tb1, sx0, sx1, st0, st1, so0, so1):
        bufs = (xb0, xb1)
        tbufs = (tb0, tb1)
        sxs = (sx0, sx1)
        sts = (st0, st1)
        sos = (so0, so1)
        cid = lax.axis_index("c")
        sid = lax.axis_index("s")
        wid = cid * _NS + sid
        tbase = wid * lpw

        def t_slice(i):
            return table_hbm.at[pl.ds(tbase + i * _CH, _CH)]

        def x_slice(i, bi):
            return x_hbm.at[pl.ds(bi * l + tbase + i * _CH, _CH)]

        def o_slice(i, bi):
            return out_hbm.at[pl.ds(bi * l + tbase + i * _CH, _CH)]

        pltpu.async_copy(t_slice(0), tbufs[0], sts[0])
        pltpu.async_copy(t_slice(1), tbufs[1], sts[1])
        pltpu.async_copy(x_slice(0, 0), bufs[0], sxs[0])

        def iter_body(i2, _):
            for ip in range(2):
                i = 2 * i2 + ip
                pltpu.make_async_copy(t_slice(i), tbufs[ip], sts[ip]).wait()
                for bi in range(b):
                    p = bi % 2
                    s = i * b + bi
                    xb = bufs[p]
                    pltpu.make_async_copy(
                        x_slice(i, bi), xb, sxs[p]).wait()
                    nbi = (bi + 1) % b
                    ni = i + (1 if bi == b - 1 else 0)

                    @pl.when(s + 1 < nsteps)
                    def _():
                        @pl.when(s >= 1)
                        def _():
                            pltpu.make_async_copy(
                                bufs[1 - p], o_slice(ni, nbi),
                                sos[1 - p]).wait()
                        pltpu.async_copy(
                            x_slice(ni, nbi), bufs[1 - p], sxs[1 - p])

                    tb = tbufs[ip]

                    @plsc.parallel_loop(0, _CH, step=1)
                    def add_body(r, xb=xb, tb=tb):
                        for c in range(nvec):
                            plsc.addupdate(
                                xb.at[r].at[pl.ds(c * _LANES, _LANES)],
                                tb[r, pl.ds(c * _LANES, _LANES)])

                    pltpu.async_copy(xb, o_slice(i, bi), sos[p])

                @pl.when(i + 2 < nch)
                def _():
                    pltpu.async_copy(t_slice(i + 2), tbufs[ip], sts[ip])
            return 0

        lax.fori_loop(0, nch // 2, iter_body, 0)
        pltpu.make_async_copy(bufs[0], o_slice(nch - 1, b - 2), sos[0]).wait()
        pltpu.make_async_copy(bufs[1], o_slice(nch - 1, b - 1), sos[1]).wait()

    return sc_add


def kernel(x, table):
    B, L, D = x.shape
    out = _make_sc_add(B, L, D)(x.reshape(B * L, D), table)
    return out.reshape(B, L, D)
